# baseline (speedup 1.0000x reference)
import functools
import jax
import jax.numpy as jnp
import numpy as np
from jax.experimental import pallas as pl
from jax.experimental.pallas import tpu as pltpu

_N0, _G0, _B, _E = 10000, 200, 50, 160000
_HID = [64, 128, 256, 512]
_H = 8


def _cdiv(a, b):
    return (a + b - 1) // b


def _mm_body(a_ref, b_ref, o_ref):
    o_ref[...] = jnp.dot(a_ref[...], b_ref[...],
                         preferred_element_type=jnp.float32)


def _mm(a, b, bm=512, bn=512):
    M, K = a.shape
    K2, N = b.shape
    assert K == K2
    bm = min(bm, max(8, M))
    bn = min(bn, max(128, N))
    gm, gn = _cdiv(M, bm), _cdiv(N, bn)
    return pl.pallas_call(
        _mm_body,
        grid=(gm, gn),
        in_specs=[
            pl.BlockSpec((bm, K), lambda i, j: (i, 0)),
            pl.BlockSpec((K, bn), lambda i, j: (0, j)),
        ],
        out_specs=pl.BlockSpec((bm, bn), lambda i, j: (i, j)),
        out_shape=jax.ShapeDtypeStruct((M, N), jnp.float32),
    )(a, b)


def _cnn_body(x_ref, w_ref, b_ref, g_ref, bb_ref, o_ref):
    h = jnp.dot(x_ref[...], w_ref[...], preferred_element_type=jnp.float32)
    h = h + b_ref[...]
    h = (h / np.sqrt(1.0 + 1e-5)) * g_ref[...] + bb_ref[...]
    o_ref[...] = jnp.where(h > 0, h, 0.01 * h)


def _cnn(x, w, b, g, bb, bm=1024):
    M, K = x.shape
    N = w.shape[1]
    gm = _cdiv(M, bm)
    vec = lambda i: (0, 0)
    return pl.pallas_call(
        _cnn_body,
        grid=(gm,),
        in_specs=[
            pl.BlockSpec((bm, K), lambda i: (i, 0)),
            pl.BlockSpec((K, N), lambda i: (0, 0)),
            pl.BlockSpec((1, N), vec),
            pl.BlockSpec((1, N), vec),
            pl.BlockSpec((1, N), vec),
        ],
        out_specs=pl.BlockSpec((bm, N), lambda i: (i, 0)),
        out_shape=jax.ShapeDtypeStruct((M, N), jnp.float32),
    )(x, w, b.reshape(1, N), g.reshape(1, N), bb.reshape(1, N))


def _readout_body(h_ref, w_ref, b_ref, o_ref):
    gx = jnp.mean(h_ref[...], axis=1)
    o_ref[...] = (jnp.dot(gx, w_ref[...], preferred_element_type=jnp.float32)
                  + b_ref[...])


def _readout(h, G, w, b):
    C = h.shape[1]
    N = w.shape[1]
    h3 = h.reshape(_B, G, C)
    return pl.pallas_call(
        _readout_body,
        grid=(1,),
        in_specs=[
            pl.BlockSpec((_B, G, C), lambda i: (0, 0, 0)),
            pl.BlockSpec((C, N), lambda i: (0, 0)),
            pl.BlockSpec((1, N), lambda i: (0, 0)),
        ],
        out_specs=pl.BlockSpec((_B, N), lambda i: (0, 0)),
        out_shape=jax.ShapeDtypeStruct((_B, N), jnp.float32),
    )(h3, w, b.reshape(1, N))


def _final_body(f_ref, fw_ref, fb_ref, e_ref, ew_ref, eb_ref, o_ref):
    a = jnp.dot(f_ref[...], fw_ref[...], preferred_element_type=jnp.float32)
    c = jnp.dot(e_ref[...], ew_ref[...], preferred_element_type=jnp.float32)
    o_ref[...] = a + c + fb_ref[...] + eb_ref[...]


def _final(fsum, fc_w, fc_b, esm, esm_w, esm_b):
    N = fc_w.shape[1]
    return pl.pallas_call(
        _final_body,
        grid=(1,),
        in_specs=[
            pl.BlockSpec(fsum.shape, lambda i: (0, 0)),
            pl.BlockSpec(fc_w.shape, lambda i: (0, 0)),
            pl.BlockSpec((1, N), lambda i: (0, 0)),
            pl.BlockSpec(esm.shape, lambda i: (0, 0)),
            pl.BlockSpec(esm_w.shape, lambda i: (0, 0)),
            pl.BlockSpec((1, N), lambda i: (0, 0)),
        ],
        out_specs=pl.BlockSpec((_B, N), lambda i: (0, 0)),
        out_shape=jax.ShapeDtypeStruct((_B, N), jnp.float32),
    )(fsum, fc_w, fc_b.reshape(1, N), esm, esm_w, esm_b.reshape(1, N))


def _att_mat(a):
    Hh, C = a.shape
    return (jnp.eye(Hh, dtype=a.dtype)[:, None, :] * a[:, :, None]).reshape(
        Hh * C, Hh)


def _gat(h, src, dst, emask, W, a_s, a_d, bias, C):
    N = h.shape[0]
    loop = jnp.arange(N, dtype=src.dtype)
    sf = jnp.concatenate([src, loop])
    df = jnp.concatenate([dst, loop])
    vm = jnp.concatenate([emask, jnp.ones((N,), bool)])
    xw2 = _mm(h, W)
    als = _mm(xw2, _att_mat(a_s))
    ald = _mm(xw2, _att_mat(a_d))
    xw = xw2.reshape(N, _H, C)
    logits = jax.nn.leaky_relu(als[sf] + ald[df], 0.2)
    logits = jnp.where(vm[:, None], logits, -1e9)
    m = jax.ops.segment_max(logits, df, num_segments=N)
    ex = jnp.exp(logits - m[df]) * vm[:, None].astype(h.dtype)
    den = jax.ops.segment_sum(ex, df, num_segments=N)
    alpha = ex / (den[df] + 1e-16)
    out = jnp.zeros((N, C), h.dtype)
    for hh in range(_H):
        out = out + jax.ops.segment_sum(
            xw[sf, hh, :] * alpha[:, hh, None], df, num_segments=N)
    return out / _H + bias


def _sag_pool(x, src, dst, emask, rel_w, rel_b, root_w, G):
    N = x.shape[0]
    k = G // 2
    p = (x @ rel_w).reshape(-1)
    agg = jax.ops.segment_sum(p[src] * emask.astype(x.dtype), dst,
                              num_segments=N)
    score = agg + rel_b.reshape(()) + (x @ root_w).reshape(-1)
    topv, topi = jax.lax.top_k(score.reshape(_B, G), k)
    perm = (topi + (jnp.arange(_B) * G)[:, None]).reshape(-1)
    x_new = x[perm] * jnp.tanh(score[perm])[:, None]
    mapping = jnp.full((N,), -1, jnp.int32).at[perm].set(
        jnp.arange(_B * k, dtype=jnp.int32))
    ns = mapping[src]
    nd = mapping[dst]
    nm = emask & (ns >= 0) & (nd >= 0)
    ns = jnp.where(nm, ns, 0)
    nd = jnp.where(nm, nd, 0)
    return x_new, ns, nd, nm


def kernel(x, esm_feature, edge_index, batch, cnn_w, cnn_b, bn_g, bn_b,
           gat_w0, att_src0, att_dst0, gat_b0, gm_w0, gm_b0,
           gat_w1, att_src1, att_dst1, gat_b1, gm_w1, gm_b1,
           gat_w2, att_src2, att_dst2, gat_b2, gm_w2, gm_b2,
           rel_w0, rel_b0, root_w0, rel_w1, rel_b1, root_w1,
           fc_w, fc_b, esm_w, esm_b):
    src = edge_index[0]
    dst = edge_index[1]
    emask = jnp.ones((_E,), bool)
    h = _cnn(x, cnn_w, cnn_b, bn_g, bn_b)
    gat_ps = [(gat_w0, att_src0, att_dst0, gat_b0),
              (gat_w1, att_src1, att_dst1, gat_b1),
              (gat_w2, att_src2, att_dst2, gat_b2)]
    gm_ps = [(gm_w0, gm_b0), (gm_w1, gm_b1), (gm_w2, gm_b2)]
    pool_ps = [(rel_w0, rel_b0, root_w0), (rel_w1, rel_b1, root_w1)]
    G = _G0
    fsum = None
    for i in range(3):
        h = _gat(h, src, dst, emask, gat_ps[i][0], gat_ps[i][1],
                 gat_ps[i][2], gat_ps[i][3], _HID[i + 1])
        f = _readout(h, G, gm_ps[i][0], gm_ps[i][1])
        fsum = f if fsum is None else fsum + f
        if i < 2:
            h = jax.nn.leaky_relu(h, 0.01)
            h, src, dst, emask = _sag_pool(
                h, src, dst, emask, pool_ps[i][0], pool_ps[i][1],
                pool_ps[i][2], G)
            G = G // 2
    return _final(fsum, fc_w, fc_b, esm_feature, esm_w, esm_b)


# speedup vs baseline: 1.0490x; 1.0490x over previous
import functools
import jax
import jax.numpy as jnp
import numpy as np
from jax import lax
from jax.experimental import pallas as pl
from jax.experimental.pallas import tpu as pltpu
from jax.experimental.pallas import tpu_sc as plsc

_N0, _G0, _B, _E = 10000, 200, 50, 160000
_HID = [64, 128, 256, 512]
_H = 8


def _cdiv(a, b):
    return (a + b - 1) // b


def _mm_body(a_ref, b_ref, o_ref):
    o_ref[...] = jnp.dot(a_ref[...], b_ref[...],
                         preferred_element_type=jnp.float32)


def _mm(a, b, bm=512, bn=512):
    M, K = a.shape
    K2, N = b.shape
    assert K == K2
    bm = min(bm, max(8, M))
    bn = min(bn, max(128, N))
    gm, gn = _cdiv(M, bm), _cdiv(N, bn)
    return pl.pallas_call(
        _mm_body,
        grid=(gm, gn),
        in_specs=[
            pl.BlockSpec((bm, K), lambda i, j: (i, 0)),
            pl.BlockSpec((K, bn), lambda i, j: (0, j)),
        ],
        out_specs=pl.BlockSpec((bm, bn), lambda i, j: (i, j)),
        out_shape=jax.ShapeDtypeStruct((M, N), jnp.float32),
    )(a, b)


def _cnn_body(x_ref, w_ref, b_ref, g_ref, bb_ref, o_ref):
    h = jnp.dot(x_ref[...], w_ref[...], preferred_element_type=jnp.float32)
    h = h + b_ref[...]
    h = (h / np.sqrt(1.0 + 1e-5)) * g_ref[...] + bb_ref[...]
    o_ref[...] = jnp.where(h > 0, h, 0.01 * h)


def _cnn(x, w, b, g, bb, bm=1024):
    M, K = x.shape
    N = w.shape[1]
    gm = _cdiv(M, bm)
    vec = lambda i: (0, 0)
    return pl.pallas_call(
        _cnn_body,
        grid=(gm,),
        in_specs=[
            pl.BlockSpec((bm, K), lambda i: (i, 0)),
            pl.BlockSpec((K, N), lambda i: (0, 0)),
            pl.BlockSpec((1, N), vec),
            pl.BlockSpec((1, N), vec),
            pl.BlockSpec((1, N), vec),
        ],
        out_specs=pl.BlockSpec((bm, N), lambda i: (i, 0)),
        out_shape=jax.ShapeDtypeStruct((M, N), jnp.float32),
    )(x, w, b.reshape(1, N), g.reshape(1, N), bb.reshape(1, N))


def _readout_body(h_ref, w_ref, b_ref, o_ref):
    gx = jnp.mean(h_ref[...], axis=1)
    o_ref[...] = (jnp.dot(gx, w_ref[...], preferred_element_type=jnp.float32)
                  + b_ref[...])


def _readout(h, G, w, b):
    C = h.shape[1]
    N = w.shape[1]
    h3 = h.reshape(_B, G, C)
    return pl.pallas_call(
        _readout_body,
        grid=(1,),
        in_specs=[
            pl.BlockSpec((_B, G, C), lambda i: (0, 0, 0)),
            pl.BlockSpec((C, N), lambda i: (0, 0)),
            pl.BlockSpec((1, N), lambda i: (0, 0)),
        ],
        out_specs=pl.BlockSpec((_B, N), lambda i: (0, 0)),
        out_shape=jax.ShapeDtypeStruct((_B, N), jnp.float32),
    )(h3, w, b.reshape(1, N))


def _final_body(f_ref, fw_ref, fb_ref, e_ref, ew_ref, eb_ref, o_ref):
    a = jnp.dot(f_ref[...], fw_ref[...], preferred_element_type=jnp.float32)
    c = jnp.dot(e_ref[...], ew_ref[...], preferred_element_type=jnp.float32)
    o_ref[...] = a + c + fb_ref[...] + eb_ref[...]


def _final(fsum, fc_w, fc_b, esm, esm_w, esm_b):
    N = fc_w.shape[1]
    return pl.pallas_call(
        _final_body,
        grid=(1,),
        in_specs=[
            pl.BlockSpec(fsum.shape, lambda i: (0, 0)),
            pl.BlockSpec(fc_w.shape, lambda i: (0, 0)),
            pl.BlockSpec((1, N), lambda i: (0, 0)),
            pl.BlockSpec(esm.shape, lambda i: (0, 0)),
            pl.BlockSpec(esm_w.shape, lambda i: (0, 0)),
            pl.BlockSpec((1, N), lambda i: (0, 0)),
        ],
        out_specs=pl.BlockSpec((_B, N), lambda i: (0, 0)),
        out_shape=jax.ShapeDtypeStruct((_B, N), jnp.float32),
    )(fsum, fc_w, fc_b.reshape(1, N), esm, esm_w, esm_b.reshape(1, N))


def _att_mat(a):
    Hh, C = a.shape
    return (jnp.eye(Hh, dtype=a.dtype)[:, None, :] * a[:, :, None]).reshape(
        Hh * C, Hh)


_CH = 512
_K = 16


@functools.lru_cache(maxsize=None)
def _make_att(EP, Np):
    TPE = EP // 32
    NCH = TPE // _CH
    mesh = plsc.VectorSubcoreMesh(core_axis_name="c", subcore_axis_name="s")

    @functools.partial(
        pl.kernel,
        out_type=[jax.ShapeDtypeStruct((8, EP), jnp.float32),
                  jax.ShapeDtypeStruct((32, 8, Np), jnp.float32)],
        mesh=mesh,
        compiler_params=pltpu.CompilerParams(needs_layout_passes=False),
        scratch_types=[
            pltpu.VMEM((_CH,), jnp.int32),
            pltpu.VMEM((_CH,), jnp.int32),
            pltpu.VMEM((_CH,), jnp.float32),
            pltpu.VMEM((_CH,), jnp.float32),
            pltpu.VMEM((Np,), jnp.float32),
            pltpu.VMEM((Np,), jnp.float32),
            pltpu.VMEM((Np,), jnp.float32),
            pltpu.VMEM((16,), jnp.float32),
        ],
    )
    def att(sf_hbm, df_hbm, vm_hbm, als_hbm, ald_hbm, gmax_hbm, zer_hbm,
            ex_hbm, den_hbm,
            sfv, dfv, vmv, exv, alsv, aldv, denv, gmv):
        c = lax.axis_index("c")
        s = lax.axis_index("s")
        wid = s * 2 + c
        pltpu.sync_copy(gmax_hbm, gmv)
        for h in range(8):
            pltpu.sync_copy(als_hbm.at[h], alsv)
            pltpu.sync_copy(ald_hbm.at[h], aldv)
            pltpu.sync_copy(zer_hbm, denv)
            gm = plsc.load_gather(gmv, [jnp.full((16,), h, jnp.int32)])

            def chunk_body(g, carry):
                base = wid * TPE + g * _CH
                pltpu.sync_copy(sf_hbm.at[pl.ds(base, _CH)], sfv)
                pltpu.sync_copy(df_hbm.at[pl.ds(base, _CH)], dfv)
                pltpu.sync_copy(vm_hbm.at[pl.ds(base, _CH)], vmv)

                def lane_body(j, cc):
                    si = sfv[pl.ds(j * 16, 16)]
                    di = dfv[pl.ds(j * 16, 16)]
                    a = plsc.load_gather(alsv, [si])
                    b = plsc.load_gather(aldv, [di])
                    l = a + b
                    l = jnp.maximum(l, 0.2 * l) - gm
                    e = jnp.exp(l) * vmv[pl.ds(j * 16, 16)]
                    exv[pl.ds(j * 16, 16)] = e
                    plsc.addupdate_scatter(denv, [di], e)
                    return cc

                lax.fori_loop(0, _CH // 16, lane_body, 0)
                pltpu.sync_copy(exv, ex_hbm.at[h, pl.ds(base, _CH)])
                return carry

            lax.fori_loop(0, NCH, chunk_body, 0)
            pltpu.sync_copy(denv, den_hbm.at[wid, h])

    return att


@functools.lru_cache(maxsize=None)
def _make_agg(EP, Np, C):
    TPE = EP // 32
    K = _K
    NCH = TPE // K
    HC = 8 * C
    SL = Np // 16
    NDUMP = SL // K
    mesh = plsc.VectorSubcoreMesh(core_axis_name="c", subcore_axis_name="s")

    @functools.partial(
        pl.kernel,
        out_type=jax.ShapeDtypeStruct((2, Np, C), jnp.float32),
        mesh=mesh,
        compiler_params=pltpu.CompilerParams(needs_layout_passes=False),
        scratch_types=[
            pltpu.VMEM((K,), jnp.int32),
            pltpu.VMEM((K,), jnp.int32),
            pltpu.VMEM((8 * K,), jnp.float32),
            pltpu.VMEM((8 * K,), jnp.float32),
            pltpu.VMEM((8 * Np,), jnp.float32),
            pltpu.VMEM((K, HC), jnp.float32),
            pltpu.VMEM((K, C), jnp.float32),
            pltpu.VMEM_SHARED((Np, C), jnp.float32),
            pltpu.SemaphoreType.DMA,
        ],
    )
    def agg(sf_hbm, df_hbm, ex_hbm, dinv_hbm, xw_hbm, zer_hbm,
            out_hbm,
            sfv, dfv, exv, alv, dnv, xwv, con, osh, sem1):
        c = lax.axis_index("c")
        s = lax.axis_index("s")
        wid = s * 2 + c
        for r in range(NDUMP):
            pltpu.sync_copy(zer_hbm, con)
            pltpu.sync_copy(con, osh.at[pl.ds(s * SL + r * K, K)])
        pltpu.sync_copy(dinv_hbm, dnv)
        plsc.subcore_barrier()

        def chunk_body(g, carry):
            base = wid * TPE + g * K
            pltpu.sync_copy(sf_hbm.at[pl.ds(base, K)], sfv)
            pltpu.sync_copy(df_hbm.at[pl.ds(base, K)], dfv)
            for h in range(8):
                pltpu.sync_copy(ex_hbm.at[h, pl.ds(base, K)],
                                exv.at[pl.ds(h * K, K)])
            cp = pltpu.async_copy(xw_hbm.at[sfv], xwv, sem1)
            di = dfv[pl.ds(0, 16)]
            for h in range(8):
                dn = plsc.load_gather(dnv, [di + (h * Np)])
                al = exv[pl.ds(h * 16, 16)] * dn
                alv[pl.ds(h * 16, 16)] = al
            cp.wait()

            def edge_body(e, cc):
                spl = [plsc.load_gather(
                    alv, [jnp.full((16,), hh * 16, jnp.int32) + e])
                    for hh in range(8)]
                for cb in range(C // 16):
                    acc = spl[0] * xwv[e, pl.ds(cb * 16, 16)]
                    for hh in range(1, 8):
                        acc = acc + spl[hh] * xwv[e,
                                                  pl.ds(hh * C + cb * 16, 16)]
                    con[e, pl.ds(cb * 16, 16)] = acc
                return cc

            lax.fori_loop(0, K, edge_body, 0)
            pltpu.sync_copy(con, osh.at[dfv], add=True)
            return carry

        lax.fori_loop(0, NCH, chunk_body, 0)
        plsc.subcore_barrier()
        for r in range(NDUMP):
            pltpu.sync_copy(osh.at[pl.ds(s * SL + r * K, K)], con)
            pltpu.sync_copy(con, out_hbm.at[c, pl.ds(s * SL + r * K, K)])

    return agg


def _gat(h, src, dst, emask, W, a_s, a_d, bias, C):
    N = h.shape[0]
    loop = jnp.arange(N, dtype=src.dtype)
    sf0 = jnp.concatenate([src, loop])
    df0 = jnp.concatenate([dst, loop])
    vm0 = jnp.concatenate([emask.astype(jnp.float32),
                           jnp.ones((N,), jnp.float32)])
    xw2 = _mm(h, W)
    als = _mm(xw2, _att_mat(a_s))
    ald = _mm(xw2, _att_mat(a_d))

    ne = _E + N
    EP = _cdiv(ne, 32 * _CH) * 32 * _CH
    pad = EP - ne
    sf = jnp.pad(sf0, (0, pad))
    df = jnp.pad(df0, (0, pad))
    vm = jnp.pad(vm0, (0, pad))
    Np = _cdiv(N, 512) * 512

    mb = jnp.max(als, axis=0) + jnp.max(ald, axis=0)
    gm = jnp.where(mb > 0, mb, 0.2 * mb)

    als_t = jnp.pad(als.T, ((0, 0), (0, Np - N)))
    ald_t = jnp.pad(ald.T, ((0, 0), (0, Np - N)))
    ex, den = _make_att(EP, Np)(
        sf, df, vm, als_t, ald_t, jnp.tile(gm, 2).astype(jnp.float32),
        jnp.zeros((Np,), jnp.float32))
    dinv = 1.0 / (den.sum(axis=0) + 1e-30)
    alpha = ex[:, :ne].T * dinv[:, df0].T
    xw = xw2.reshape(N, _H, C)
    out = jnp.zeros((N, C), h.dtype)
    for hh in range(_H):
        out = out + jax.ops.segment_sum(
            xw[sf0, hh, :] * alpha[:, hh, None], df0, num_segments=N)
    return out / _H + bias


def _sag_pool(x, src, dst, emask, rel_w, rel_b, root_w, G):
    N = x.shape[0]
    k = G // 2
    p = (x @ rel_w).reshape(-1)
    agg = jax.ops.segment_sum(p[src] * emask.astype(x.dtype), dst,
                              num_segments=N)
    score = agg + rel_b.reshape(()) + (x @ root_w).reshape(-1)
    topv, topi = jax.lax.top_k(score.reshape(_B, G), k)
    perm = (topi + (jnp.arange(_B) * G)[:, None]).reshape(-1)
    x_new = x[perm] * jnp.tanh(score[perm])[:, None]
    mapping = jnp.full((N,), -1, jnp.int32).at[perm].set(
        jnp.arange(_B * k, dtype=jnp.int32))
    ns = mapping[src]
    nd = mapping[dst]
    nm = emask & (ns >= 0) & (nd >= 0)
    ns = jnp.where(nm, ns, 0)
    nd = jnp.where(nm, nd, 0)
    return x_new, ns, nd, nm


def kernel(x, esm_feature, edge_index, batch, cnn_w, cnn_b, bn_g, bn_b,
           gat_w0, att_src0, att_dst0, gat_b0, gm_w0, gm_b0,
           gat_w1, att_src1, att_dst1, gat_b1, gm_w1, gm_b1,
           gat_w2, att_src2, att_dst2, gat_b2, gm_w2, gm_b2,
           rel_w0, rel_b0, root_w0, rel_w1, rel_b1, root_w1,
           fc_w, fc_b, esm_w, esm_b):
    src = edge_index[0]
    dst = edge_index[1]
    emask = jnp.ones((_E,), bool)
    h = _cnn(x, cnn_w, cnn_b, bn_g, bn_b)
    gat_ps = [(gat_w0, att_src0, att_dst0, gat_b0),
              (gat_w1, att_src1, att_dst1, gat_b1),
              (gat_w2, att_src2, att_dst2, gat_b2)]
    gm_ps = [(gm_w0, gm_b0), (gm_w1, gm_b1), (gm_w2, gm_b2)]
    pool_ps = [(rel_w0, rel_b0, root_w0), (rel_w1, rel_b1, root_w1)]
    G = _G0
    fsum = None
    for i in range(3):
        h = _gat(h, src, dst, emask, gat_ps[i][0], gat_ps[i][1],
                 gat_ps[i][2], gat_ps[i][3], _HID[i + 1])
        f = _readout(h, G, gm_ps[i][0], gm_ps[i][1])
        fsum = f if fsum is None else fsum + f
        if i < 2:
            h = jax.nn.leaky_relu(h, 0.01)
            h, src, dst, emask = _sag_pool(
                h, src, dst, emask, pool_ps[i][0], pool_ps[i][1],
                pool_ps[i][2], G)
            G = G // 2
    return _final(fsum, fc_w, fc_b, esm_feature, esm_w, esm_b)
